# baseline (device time: 31686 ns/iter reference)
import jax
import jax.numpy as jnp
from jax import lax
from jax.experimental import pallas as pl
from jax.experimental.pallas import tpu as pltpu

N_DEV = 4
N_EXP = 8
E_PER = 2


def kernel(x, router_W, route_idx, expert_W):
    m, d = x.shape
    _, _, hdim = expert_W.shape

    def body(x_ref, rw_ref, idx_ref, ew_ref, out_ref, ew_all, send_sems, recv_sems):
        my_pos = lax.axis_index("i")
        left = lax.rem(my_pos + N_DEV - 1, N_DEV)
        right = lax.rem(my_pos + 1, N_DEV)

        barrier_sem = pltpu.get_barrier_semaphore()
        for nbr in (left, right):
            pl.semaphore_signal(
                barrier_sem, inc=1,
                device_id=(nbr,), device_id_type=pl.DeviceIdType.MESH,
            )
        pl.semaphore_wait(barrier_sem, 2)

        ew_all[0] = ew_ref[...].astype(jnp.bfloat16)

        for h in range(N_DEV - 1):
            rdma = pltpu.make_async_remote_copy(
                src_ref=ew_all.at[h],
                dst_ref=ew_all.at[h + 1],
                send_sem=send_sems.at[h],
                recv_sem=recv_sems.at[h],
                device_id=(right,),
                device_id_type=pl.DeviceIdType.MESH,
            )
            rdma.start()
            rdma.wait()

        xf = x_ref[...]
        scores = jnp.dot(xf, rw_ref[...], preferred_element_type=jnp.float32)
        smax = jnp.max(scores, axis=1, keepdims=True)
        iota = lax.broadcasted_iota(jnp.int32, (m, N_EXP), 1)
        sel = (idx_ref[:, 0:1] == iota) | (idx_ref[:, 1:2] == iota)
        p = jnp.where(sel, jnp.exp(scores - smax), 0.0)
        w = p / jnp.sum(p, axis=1, keepdims=True)

        acc = jnp.zeros((m, hdim), jnp.float32)
        for s in range(N_DEV):
            src_dev = lax.rem(my_pos - s + N_DEV, N_DEV)
            for k in range(E_PER):
                eid = src_dev * E_PER + k
                g = jnp.sum(jnp.where(iota == eid, w, 0.0), axis=1, keepdims=True)
                xg = (xf * g).astype(jnp.bfloat16)
                acc = acc + jnp.dot(
                    xg, ew_all[s, k], preferred_element_type=jnp.float32
                )
        out_ref[...] = acc

    return pl.pallas_call(
        body,
        out_shape=jax.ShapeDtypeStruct((m, hdim), jnp.float32),
        in_specs=[pl.BlockSpec(memory_space=pltpu.VMEM)] * 4,
        out_specs=pl.BlockSpec(memory_space=pltpu.VMEM),
        scratch_shapes=[
            pltpu.VMEM((N_DEV, E_PER, d, hdim), jnp.bfloat16),
            pltpu.SemaphoreType.DMA((N_DEV - 1,)),
            pltpu.SemaphoreType.DMA((N_DEV - 1,)),
        ],
        compiler_params=pltpu.CompilerParams(collective_id=0),
    )(x, router_W, route_idx, expert_W)


# device time: 20010 ns/iter; 1.5835x vs baseline; 1.5835x over previous
import jax
import jax.numpy as jnp
from jax import lax
from jax.experimental import pallas as pl
from jax.experimental.pallas import tpu as pltpu

N_DEV = 4
N_EXP = 8
E_PER = 2


def kernel(x, router_W, route_idx, expert_W):
    m, d = x.shape
    _, _, hdim = expert_W.shape

    def body(x_ref, rw_ref, idx_ref, ew_ref, out_ref, ew_all, send_sems, recv_sems):
        my_pos = lax.axis_index("i")
        left = lax.rem(my_pos + N_DEV - 1, N_DEV)
        right = lax.rem(my_pos + 1, N_DEV)

        barrier_sem = pltpu.get_barrier_semaphore()
        for nbr in (left, right):
            pl.semaphore_signal(
                barrier_sem, inc=1,
                device_id=(nbr,), device_id_type=pl.DeviceIdType.MESH,
            )
        pl.semaphore_wait(barrier_sem, 2)

        ew_all[0] = ew_ref[...].astype(jnp.bfloat16)

        def copy(src_at, dst_at, sem_idx, target):
            return pltpu.make_async_remote_copy(
                src_ref=src_at,
                dst_ref=dst_at,
                send_sem=send_sems.at[sem_idx],
                recv_sem=recv_sems.at[sem_idx],
                device_id=(target,),
                device_id_type=pl.DeviceIdType.MESH,
            )

        a1 = copy(ew_all.at[0], ew_all.at[1], 0, right)
        a2 = copy(ew_all.at[0], ew_all.at[3], 1, left)
        a1.start()
        a2.start()

        xf = x_ref[...]
        scores = jnp.dot(xf, rw_ref[...], preferred_element_type=jnp.float32)
        smax = jnp.max(scores, axis=1, keepdims=True)
        iota = lax.broadcasted_iota(jnp.int32, (m, N_EXP), 1)
        sel = (idx_ref[:, 0:1] == iota) | (idx_ref[:, 1:2] == iota)
        p = jnp.where(sel, jnp.exp(scores - smax), 0.0)
        w = p / jnp.sum(p, axis=1, keepdims=True)

        def contrib(slot, src_dev, acc):
            for k in range(E_PER):
                eid = src_dev * E_PER + k
                g = jnp.sum(jnp.where(iota == eid, w, 0.0), axis=1, keepdims=True)
                xg = (xf * g).astype(jnp.bfloat16)
                acc = acc + jnp.dot(
                    xg, ew_all[slot, k], preferred_element_type=jnp.float32
                )
            return acc

        acc = jnp.zeros((m, hdim), jnp.float32)
        acc = contrib(0, my_pos, acc)

        a1.wait_recv()
        b1 = copy(ew_all.at[1, 0], ew_all.at[2, 0], 2, right)
        b1.start()
        acc = contrib(1, lax.rem(my_pos + N_DEV - 1, N_DEV), acc)

        a2.wait_recv()
        b2 = copy(ew_all.at[3, 1], ew_all.at[2, 1], 3, left)
        b2.start()
        acc = contrib(3, lax.rem(my_pos + 1, N_DEV), acc)

        b1.wait_recv()
        b2.wait_recv()
        acc = contrib(2, lax.rem(my_pos + 2, N_DEV), acc)

        a1.wait_send()
        a2.wait_send()
        b1.wait_send()
        b2.wait_send()

        out_ref[...] = acc

    return pl.pallas_call(
        body,
        out_shape=jax.ShapeDtypeStruct((m, hdim), jnp.float32),
        in_specs=[pl.BlockSpec(memory_space=pltpu.VMEM)] * 4,
        out_specs=pl.BlockSpec(memory_space=pltpu.VMEM),
        scratch_shapes=[
            pltpu.VMEM((N_DEV, E_PER, d, hdim), jnp.bfloat16),
            pltpu.SemaphoreType.DMA((4,)),
            pltpu.SemaphoreType.DMA((4,)),
        ],
        compiler_params=pltpu.CompilerParams(collective_id=0),
    )(x, router_W, route_idx, expert_W)


# device time: 18475 ns/iter; 1.7151x vs baseline; 1.0831x over previous
import jax
import jax.numpy as jnp
from jax import lax
from jax.experimental import pallas as pl
from jax.experimental.pallas import tpu as pltpu

N_DEV = 4
N_EXP = 8
E_PER = 2


def kernel(x, router_W, route_idx, expert_W):
    m, d = x.shape
    _, _, hdim = expert_W.shape

    def body(x_ref, rw_ref, idx_ref, ew_ref, out_ref, ew_all, send_sems, recv_sems):
        my_pos = lax.axis_index("i")
        left = lax.rem(my_pos + N_DEV - 1, N_DEV)
        right = lax.rem(my_pos + 1, N_DEV)

        barrier_sem = pltpu.get_barrier_semaphore()
        for nbr in (left, right):
            pl.semaphore_signal(
                barrier_sem, inc=1,
                device_id=(nbr,), device_id_type=pl.DeviceIdType.MESH,
            )
        pl.semaphore_wait(barrier_sem, 2)

        ew_all[0] = ew_ref[...].astype(jnp.bfloat16)

        def copy(src_at, dst_at, sem_idx, target):
            return pltpu.make_async_remote_copy(
                src_ref=src_at,
                dst_ref=dst_at,
                send_sem=send_sems.at[sem_idx],
                recv_sem=recv_sems.at[sem_idx],
                device_id=(target,),
                device_id_type=pl.DeviceIdType.MESH,
            )

        a_r0 = copy(ew_all.at[0, 0], ew_all.at[1, 0], 0, right)
        a_l1 = copy(ew_all.at[0, 1], ew_all.at[3, 1], 1, left)
        a_r1 = copy(ew_all.at[0, 1], ew_all.at[1, 1], 2, right)
        a_l0 = copy(ew_all.at[0, 0], ew_all.at[3, 0], 3, left)
        for r in (a_r0, a_l1, a_r1, a_l0):
            r.start()

        xf = x_ref[...]
        scores = jnp.dot(xf, rw_ref[...], preferred_element_type=jnp.float32)
        smax = jnp.max(scores, axis=1, keepdims=True)
        iota = lax.broadcasted_iota(jnp.int32, (m, N_EXP), 1)
        sel = (idx_ref[:, 0:1] == iota) | (idx_ref[:, 1:2] == iota)
        p = jnp.where(sel, jnp.exp(scores - smax), 0.0)
        w = p / jnp.sum(p, axis=1, keepdims=True)

        def contrib(slot, k, src_dev, acc):
            eid = src_dev * E_PER + k
            g = jnp.sum(jnp.where(iota == eid, w, 0.0), axis=1, keepdims=True)
            xg = (xf * g).astype(jnp.bfloat16)
            return acc + jnp.dot(
                xg, ew_all[slot, k], preferred_element_type=jnp.float32
            )

        acc = jnp.zeros((m, hdim), jnp.float32)
        acc = contrib(0, 0, my_pos, acc)
        acc = contrib(0, 1, my_pos, acc)

        a_r0.wait_recv()
        b_r = copy(ew_all.at[1, 0], ew_all.at[2, 0], 4, right)
        b_r.start()
        acc = contrib(1, 0, left, acc)

        a_l1.wait_recv()
        b_l = copy(ew_all.at[3, 1], ew_all.at[2, 1], 5, left)
        b_l.start()
        acc = contrib(3, 1, right, acc)

        a_r1.wait_recv()
        acc = contrib(1, 1, left, acc)
        a_l0.wait_recv()
        acc = contrib(3, 0, right, acc)

        opp = lax.rem(my_pos + 2, N_DEV)
        b_r.wait_recv()
        acc = contrib(2, 0, opp, acc)
        b_l.wait_recv()
        acc = contrib(2, 1, opp, acc)

        for r in (a_r0, a_l1, a_r1, a_l0, b_r, b_l):
            r.wait_send()

        out_ref[...] = acc

    return pl.pallas_call(
        body,
        out_shape=jax.ShapeDtypeStruct((m, hdim), jnp.float32),
        in_specs=[pl.BlockSpec(memory_space=pltpu.VMEM)] * 4,
        out_specs=pl.BlockSpec(memory_space=pltpu.VMEM),
        scratch_shapes=[
            pltpu.VMEM((N_DEV, E_PER, d, hdim), jnp.bfloat16),
            pltpu.SemaphoreType.DMA((6,)),
            pltpu.SemaphoreType.DMA((6,)),
        ],
        compiler_params=pltpu.CompilerParams(collective_id=0),
    )(x, router_W, route_idx, expert_W)


# device time: 17617 ns/iter; 1.7986x vs baseline; 1.0487x over previous
import jax
import jax.numpy as jnp
from jax import lax
from jax.experimental import pallas as pl
from jax.experimental.pallas import tpu as pltpu

N_DEV = 4
N_EXP = 8
E_PER = 2


def kernel(x, router_W, route_idx, expert_W):
    m, d = x.shape
    _, _, hdim = expert_W.shape

    def body(x_ref, rwt_ref, idx_ref, ew_ref, out_ref, ew_all,
             send_sems, recv_sems):
        my_pos = lax.axis_index("i")
        left = lax.rem(my_pos + N_DEV - 1, N_DEV)
        right = lax.rem(my_pos + 1, N_DEV)

        barrier_sem = pltpu.get_barrier_semaphore()
        for nbr in (left, right):
            pl.semaphore_signal(
                barrier_sem, inc=1,
                device_id=(nbr,), device_id_type=pl.DeviceIdType.MESH,
            )
        pl.semaphore_wait(barrier_sem, 2)

        def copy(src_at, dst_at, sem_idx, target):
            return pltpu.make_async_remote_copy(
                src_ref=src_at,
                dst_ref=dst_at,
                send_sem=send_sems.at[sem_idx],
                recv_sem=recv_sems.at[sem_idx],
                device_id=(target,),
                device_id_type=pl.DeviceIdType.MESH,
            )

        a_r0 = copy(ew_ref.at[0], ew_all.at[1, 0], 0, right)
        a_l1 = copy(ew_ref.at[1], ew_all.at[3, 1], 1, left)
        a_r1 = copy(ew_ref.at[1], ew_all.at[1, 1], 2, right)
        a_l0 = copy(ew_ref.at[0], ew_all.at[3, 0], 3, left)
        for r in (a_r0, a_l1, a_r1, a_l0):
            r.start()

        xf = x_ref[...]
        scores = lax.dot_general(
            xf, rwt_ref[...],
            dimension_numbers=(((1,), (1,)), ((), ())),
            preferred_element_type=jnp.float32,
        )
        smax = jnp.max(scores, axis=1, keepdims=True)
        iota = lax.broadcasted_iota(jnp.int32, (m, N_EXP), 1)
        sel = (idx_ref[:, 0:1] == iota) | (idx_ref[:, 1:2] == iota)
        p = jnp.where(sel, jnp.exp(scores - smax), 0.0)
        w = p / jnp.sum(p, axis=1, keepdims=True)

        opp = lax.rem(my_pos + 2, N_DEV)
        slot_src = {0: my_pos, 1: left, 2: opp, 3: right}

        def gated_x(slot, k):
            eid = slot_src[slot] * E_PER + k
            g = jnp.sum(jnp.where(iota == eid, w, 0.0), axis=1, keepdims=True)
            return xf * g.astype(jnp.bfloat16)

        xg = {(s, k): gated_x(s, k) for s in range(N_DEV) for k in range(E_PER)}

        def contrib(slot, k, acc):
            rhs = ew_ref[k] if slot == 0 else ew_all[slot, k]
            return acc + jnp.dot(
                xg[(slot, k)], rhs, preferred_element_type=jnp.float32
            )

        acc = jnp.zeros((m, hdim), jnp.float32)
        acc = contrib(0, 0, acc)
        acc = contrib(0, 1, acc)

        a_r0.wait_recv()
        b_r = copy(ew_all.at[1, 0], ew_all.at[2, 0], 4, right)
        b_r.start()
        acc = contrib(1, 0, acc)

        a_l1.wait_recv()
        b_l = copy(ew_all.at[3, 1], ew_all.at[2, 1], 5, left)
        b_l.start()
        acc = contrib(3, 1, acc)

        a_r1.wait_recv()
        acc = contrib(1, 1, acc)
        a_l0.wait_recv()
        acc = contrib(3, 0, acc)

        b_r.wait_recv()
        acc = contrib(2, 0, acc)
        b_l.wait_recv()
        acc = contrib(2, 1, acc)

        for r in (a_r0, a_l1, a_r1, a_l0, b_r, b_l):
            r.wait_send()

        out_ref[...] = acc.astype(jnp.bfloat16)

    xb = x.astype(jnp.bfloat16)
    ewb = expert_W.astype(jnp.bfloat16)
    rw_t = jnp.transpose(router_W).astype(jnp.bfloat16)

    return pl.pallas_call(
        body,
        out_shape=jax.ShapeDtypeStruct((m, hdim), jnp.bfloat16),
        in_specs=[pl.BlockSpec(memory_space=pltpu.VMEM)] * 4,
        out_specs=pl.BlockSpec(memory_space=pltpu.VMEM),
        scratch_shapes=[
            pltpu.VMEM((N_DEV, E_PER, d, hdim), jnp.bfloat16),
            pltpu.SemaphoreType.DMA((6,)),
            pltpu.SemaphoreType.DMA((6,)),
        ],
        compiler_params=pltpu.CompilerParams(collective_id=0),
    )(xb, rw_t, route_idx, ewb)
